# baseline (speedup 1.0000x reference)
import functools

import jax
import jax.numpy as jnp
from jax import lax
from jax.experimental import pallas as pl
from jax.experimental.pallas import tpu as pltpu

_BINS = 30
_SCALE = 30 - 0.0001
_LANES = 128
_BR = 256
_CH = 8
_LEVELS = 12


def _bins_of(x, t):
    g = jnp.abs(jax.nn.sigmoid(x) - t)
    return jnp.floor(g * _SCALE).astype(jnp.int32)


def _csa(a, b, c):
    u = a ^ b
    return u ^ c, (a & b) | (u & c)


def _fused_kernel(x_ref, t_ref, out_ref, planes_ref, l_ref, b_ref,
                  *, nblocks, tot):
    i = pl.program_id(0)

    @pl.when(i == 0)
    def _init():
        planes_ref[...] = jnp.zeros_like(planes_ref)

    @pl.when(i < nblocks)
    def _phase_a():
        x = x_ref[...]
        t = t_ref[...]
        bb = _bins_of(x, t)
        v = jnp.left_shift(jnp.int32(1), bb)

        vals = {0: [v[k * _CH:(k + 1) * _CH, :] for k in range(_BR // _CH)]}
        j = 0
        while j in vals:
            lv = vals[j]
            carries = []
            while len(lv) >= 3:
                s, co = _csa(lv.pop(), lv.pop(), lv.pop())
                lv.append(s)
                carries.append(co)
            if len(lv) == 2:
                a0, a1 = lv
                lv = [a0 ^ a1]
                carries.append(a0 & a1)
            if carries:
                vals[j + 1] = carries
            if lv:
                carry = lv[0]
                for lvl in range(j, _LEVELS):
                    old = planes_ref[lvl]
                    planes_ref[lvl] = old ^ carry
                    carry = old & carry
            j += 1

        lval = jnp.maximum(x, 0.0) - x * t + jnp.log1p(jnp.exp(-jnp.abs(x)))
        l_ref[pl.ds(i * _BR, _BR), :] = lval.astype(jnp.bfloat16)
        b_ref[pl.ds(i * _BR, _BR), :] = bb.astype(jnp.int8)

    @pl.when(i == nblocks)
    def _phase_b():
        li = lax.broadcasted_iota(jnp.int32, (1, _LANES), 1)
        vec = jnp.zeros((1, _LANES), jnp.float32)
        for k in range(_BINS):
            c = jnp.float32(0.0)
            for lvl in range(_LEVELS):
                bits = (planes_ref[lvl] >> k) & 1
                c = c + jnp.float32(1 << lvl) * jnp.sum(bits).astype(jnp.float32)
            vec = vec + jnp.where(li == k, c, 0.0)
        ne = jnp.sum(jnp.where((li < _BINS) & (vec > 0), 1.0, 0.0))
        beta = tot / jnp.clip(vec * ne, 0.0001, None)
        cols = l_ref.shape[1]
        tab = jnp.broadcast_to(beta[:, :32], (_BR, 32))

        def body(j, _):
            lval = l_ref[pl.ds(j * _BR, _BR), :].astype(jnp.float32)
            bb = b_ref[pl.ds(j * _BR, _BR), :].astype(jnp.int32)
            w = jnp.take_along_axis(tab, bb, axis=1)
            out_ref[pl.ds(j * _BR, _BR)] = jnp.mean(w * lval, axis=1)
            return 0

        lax.fori_loop(0, nblocks, body, 0, unroll=False)


def kernel(logits, target):
    rows, cols = logits.shape
    nblocks = rows // _BR
    tot = float(logits.size)

    def in_idx(i):
        return (jnp.where(i < nblocks, i, nblocks - 1), 0)

    return pl.pallas_call(
        functools.partial(_fused_kernel, nblocks=nblocks, tot=tot),
        grid=(nblocks + 1,),
        in_specs=[
            pl.BlockSpec((_BR, cols), in_idx),
            pl.BlockSpec((_BR, cols), in_idx),
        ],
        out_specs=pl.BlockSpec((rows,), lambda i: (0,)),
        out_shape=jax.ShapeDtypeStruct((rows,), jnp.float32),
        scratch_shapes=[
            pltpu.VMEM((_LEVELS, _CH, cols), jnp.int32),
            pltpu.VMEM((rows, cols), jnp.bfloat16),
            pltpu.VMEM((rows, cols), jnp.int8),
        ],
        compiler_params=pltpu.CompilerParams(
            dimension_semantics=("arbitrary",),
        ),
    )(logits, target)
